# Initial kernel scaffold; baseline (speedup 1.0000x reference)
import jax
import jax.numpy as jnp
from jax.experimental import pallas as pl


def kernel(x, pos_edge_index, neg_edge_index, W_pos_base, b_pos_base, W_neg_base, b_neg_base, W_pos_deep, b_pos_deep, W_neg_deep, b_neg_deep, W_ih, W_hh, b_ih, b_hh):
    raise NotImplementedError("write your pallas kernel here")


# speedup vs baseline: 6.3597x; 6.3597x over previous
import functools

import jax
import jax.numpy as jnp
from jax import lax
from jax.experimental import pallas as pl
from jax.experimental.pallas import tpu as pltpu
from jax.experimental.pallas import tpu_sc as plsc

NUM_CORES = 2
NUM_TILES = 16
CHUNK = 128
CNT_W = 16
R_BLOCK = 1024

_f32 = jnp.float32
_i32 = jnp.int32


@functools.cache
def _make_seg_sum(n_pad: int, d: int, chunks: int, with_counts: bool):
    rpt = n_pad // NUM_TILES
    ib = 16
    nblocks = chunks // ib

    mesh = plsc.VectorSubcoreMesh(
        core_axis_name="c", subcore_axis_name="s",
        num_cores=NUM_CORES, num_subcores=NUM_TILES)

    out_type = [jax.ShapeDtypeStruct((NUM_CORES, n_pad, d), _f32)]
    scratch = [
        pltpu.VMEM((ib, CHUNK), _i32),
        pltpu.VMEM((ib, CHUNK), _i32),
        pltpu.VMEM((CHUNK, d), _f32),
        pltpu.SemaphoreType.DMA,
        pltpu.VMEM_SHARED((n_pad, d), _f32),
    ]
    if with_counts:
        out_type.append(jax.ShapeDtypeStruct((NUM_CORES, n_pad, CNT_W), _f32))
        scratch += [
            pltpu.VMEM((CHUNK, CNT_W), _f32),
            pltpu.VMEM_SHARED((n_pad, CNT_W), _f32),
        ]

    @functools.partial(
        pl.kernel, out_type=tuple(out_type), mesh=mesh,
        scratch_types=scratch,
        compiler_params=pltpu.CompilerParams(use_tc_tiling_on_sc=False))
    def body(table_hbm, src_hbm, dst_hbm, zrow_hbm, *rest):
        if with_counts:
            (zcnt_hbm, ones_hbm, out_sum, out_cnt,
             src_v, dst_v, rows_v, sem, acc_sh, ones_v, cnt_sh) = rest
        else:
            out_sum, src_v, dst_v, rows_v, sem, acc_sh = rest

        cid = lax.axis_index("c")
        sid = lax.axis_index("s")

        pltpu.sync_copy(zrow_hbm, acc_sh.at[pl.ds(sid * rpt, rpt)])
        if with_counts:
            pltpu.sync_copy(zcnt_hbm, cnt_sh.at[pl.ds(sid * rpt, rpt)])
            pltpu.sync_copy(ones_hbm, ones_v)

        plsc.subcore_barrier()

        def block_body(b, carry):
            base = sid * chunks + b * ib
            pltpu.sync_copy(src_hbm.at[cid, pl.ds(base, ib)], src_v)
            pltpu.sync_copy(dst_hbm.at[cid, pl.ds(base, ib)], dst_v)

            def chunk_body(c, carry2):
                pltpu.async_copy(table_hbm.at[src_v.at[c]], rows_v, sem).wait()
                pltpu.sync_copy(rows_v, acc_sh.at[dst_v.at[c]], add=True)
                if with_counts:
                    pltpu.sync_copy(ones_v, cnt_sh.at[dst_v.at[c]], add=True)
                return carry2

            return lax.fori_loop(0, ib, chunk_body, carry)

        lax.fori_loop(0, nblocks, block_body, 0)
        plsc.subcore_barrier()

        sl = pl.ds(sid * rpt, rpt)
        pltpu.sync_copy(acc_sh.at[sl], out_sum.at[cid, sl])
        if with_counts:
            pltpu.sync_copy(cnt_sh.at[sl], out_cnt.at[cid, sl])

    return body


def _pad_edges(edge_index, e_pad, dummy_dst):
    e = edge_index.shape[1]
    src = jnp.concatenate(
        [edge_index[0], jnp.zeros((e_pad - e,), _i32)]).reshape(-1, CHUNK)
    dst = jnp.concatenate(
        [edge_index[1], jnp.full((e_pad - e,), dummy_dst, _i32)]
    ).reshape(-1, CHUNK)
    return src, dst


def _row_spec(r, cols):
    return pl.BlockSpec((r, cols), lambda i: (i, 0))


def _pick_spec(which, r, cols):
    return pl.BlockSpec((1, r, cols), lambda i, w=which: (w, i, 0))


def _full_spec(shape):
    nd = len(shape)
    return pl.BlockSpec(shape, lambda i: (0,) * nd)


def _recip(cnt_ref):
    return 1.0 / jnp.maximum(cnt_ref[0][:, :1], 1.0)


def _base_dense_body(sum_p, sum_n, cnt_p, cnt_n, x, wp, wn, bp, bn, out):
    agg_p = sum_p[0] * _recip(cnt_p)
    agg_n = sum_n[0] * _recip(cnt_n)
    xb = x[...]
    fp = jnp.concatenate([agg_p, xb], axis=1)
    fn = jnp.concatenate([agg_n, xb], axis=1)
    hp = jnp.tanh(jnp.dot(fp, wp[...], preferred_element_type=_f32) + bp[...])
    hn = jnp.tanh(jnp.dot(fn, wn[...], preferred_element_type=_f32) + bn[...])
    out[...] = jnp.concatenate([hp, hn], axis=1)


def _deep_dense_body(s_p, s_n, cnt_p, cnt_n, hcat, wp, wn, bp, bn, out):
    rp = _recip(cnt_p)
    rn = _recip(cnt_n)
    sp = s_p[0]
    sn = s_n[0]
    hb = hcat[...]
    hp, hn = hb[:, :32], hb[:, 32:]
    feat = jnp.concatenate([
        sp[:, :32] * rp,
        sn[:, 32:] * rn,
        sp[:, 32:] * rp,
        sn[:, :32] * rn,
        hp, hn, 0.5 * (hp + hn),
    ], axis=1)
    np_ = jnp.tanh(jnp.dot(feat, wp[...], preferred_element_type=_f32) + bp[...])
    nn_ = jnp.tanh(jnp.dot(feat, wn[...], preferred_element_type=_f32) + bn[...])
    out[...] = jnp.concatenate([np_, nn_], axis=1)


def _lstm_body(emb, w_ih_t, w_hh_t, bias, out):
    eb = emb[...]
    cells = w_ih_t.shape[0]
    h = jnp.zeros((eb.shape[0], 64), _f32)
    c = jnp.zeros((eb.shape[0], 64), _f32)
    for i in range(cells):
        gates = (jnp.dot(eb, w_ih_t[i], preferred_element_type=_f32)
                 + jnp.dot(h, w_hh_t[i], preferred_element_type=_f32)
                 + bias[i])
        ig = jax.nn.sigmoid(gates[:, :64])
        fg = jax.nn.sigmoid(gates[:, 64:128])
        gg = jnp.tanh(gates[:, 128:192])
        og = jax.nn.sigmoid(gates[:, 192:256])
        c = fg * c + ig * gg
        h = og * jnp.tanh(c)
    out[...] = h


def kernel(x, pos_edge_index, neg_edge_index, W_pos_base, b_pos_base,
           W_neg_base, b_neg_base, W_pos_deep, b_pos_deep, W_neg_deep,
           b_neg_deep, W_ih, W_hh, b_ih, b_hh):
    n, d_feat = x.shape
    e = pos_edge_index.shape[1]
    hid = W_pos_base.shape[1]
    dcat = 2 * hid

    n_pad = ((n + R_BLOCK - 1) // R_BLOCK) * R_BLOCK
    grid = n_pad // R_BLOCK
    epc = NUM_TILES * CHUNK
    chunks = (((e + epc - 1) // epc + 7) // 8) * 8
    e_pad = chunks * epc

    src_p, dst_p = _pad_edges(pos_edge_index, e_pad, n_pad - 1)
    src_n, dst_n = _pad_edges(neg_edge_index, e_pad, n_pad - 1)
    src_all = jnp.stack([src_p, src_n])
    dst_all = jnp.stack([dst_p, dst_n])

    rpt = n_pad // NUM_TILES
    zrow_base = jnp.zeros((rpt, d_feat), _f32)
    zrow_deep = jnp.zeros((rpt, dcat), _f32)
    zcnt = jnp.zeros((rpt, CNT_W), _f32)
    ones = jnp.ones((CHUNK, CNT_W), _f32)

    x_pad = jnp.pad(x, ((0, n_pad - n), (0, 0)))

    seg_base = _make_seg_sum(n_pad, d_feat, chunks, True)
    sums, cnts = seg_base(x, src_all, dst_all, zrow_base, zcnt, ones)

    bp = b_pos_base.reshape(1, hid)
    bn = b_neg_base.reshape(1, hid)
    hcat = pl.pallas_call(
        _base_dense_body,
        grid=(grid,),
        in_specs=[
            _pick_spec(0, R_BLOCK, d_feat), _pick_spec(1, R_BLOCK, d_feat),
            _pick_spec(0, R_BLOCK, CNT_W), _pick_spec(1, R_BLOCK, CNT_W),
            _row_spec(R_BLOCK, d_feat),
            _full_spec(W_pos_base.shape), _full_spec(W_neg_base.shape),
            _full_spec(bp.shape), _full_spec(bn.shape),
        ],
        out_specs=_row_spec(R_BLOCK, dcat),
        out_shape=jax.ShapeDtypeStruct((n_pad, dcat), _f32),
    )(sums, sums, cnts, cnts, x_pad, W_pos_base, W_neg_base, bp, bn)

    seg_deep = _make_seg_sum(n_pad, dcat, chunks, False)
    for i in range(W_pos_deep.shape[0]):
        (dsums,) = seg_deep(hcat, src_all, dst_all, zrow_deep)
        bpi = b_pos_deep[i].reshape(1, hid)
        bni = b_neg_deep[i].reshape(1, hid)
        hcat = pl.pallas_call(
            _deep_dense_body,
            grid=(grid,),
            in_specs=[
                _pick_spec(0, R_BLOCK, dcat), _pick_spec(1, R_BLOCK, dcat),
                _pick_spec(0, R_BLOCK, CNT_W), _pick_spec(1, R_BLOCK, CNT_W),
                _row_spec(R_BLOCK, dcat),
                _full_spec(W_pos_deep[i].shape),
                _full_spec(W_neg_deep[i].shape),
                _full_spec(bpi.shape), _full_spec(bni.shape),
            ],
            out_specs=_row_spec(R_BLOCK, dcat),
            out_shape=jax.ShapeDtypeStruct((n_pad, dcat), _f32),
        )(dsums, dsums, cnts, cnts, hcat, W_pos_deep[i], W_neg_deep[i],
          bpi, bni)

    w_ih_t = W_ih.transpose(0, 2, 1)
    w_hh_t = W_hh.transpose(0, 2, 1)
    bias = (b_ih + b_hh).reshape(W_ih.shape[0], 1, W_ih.shape[1])
    hx = pl.pallas_call(
        _lstm_body,
        grid=(grid,),
        in_specs=[
            _row_spec(R_BLOCK, dcat),
            _full_spec(w_ih_t.shape), _full_spec(w_hh_t.shape),
            _full_spec(bias.shape),
        ],
        out_specs=_row_spec(R_BLOCK, 64),
        out_shape=jax.ShapeDtypeStruct((n_pad, 64), _f32),
    )(hcat, w_ih_t, w_hh_t, bias)

    return hx[:n]


# speedup vs baseline: 6.6502x; 1.0457x over previous
import functools

import jax
import jax.numpy as jnp
from jax import lax
from jax.experimental import pallas as pl
from jax.experimental.pallas import tpu as pltpu
from jax.experimental.pallas import tpu_sc as plsc

NUM_CORES = 2
NUM_TILES = 16
CHUNK = 64
NBUF = 4
IB = 16
CNT_W = 16
R_BLOCK = 1024

_f32 = jnp.float32
_i32 = jnp.int32


@functools.cache
def _make_seg_sum(n_pad: int, d: int, chunks: int, with_counts: bool):
    rpt = n_pad // NUM_TILES
    nblocks = chunks // IB
    groups = IB // NBUF

    mesh = plsc.VectorSubcoreMesh(
        core_axis_name="c", subcore_axis_name="s",
        num_cores=NUM_CORES, num_subcores=NUM_TILES)

    out_type = [jax.ShapeDtypeStruct((NUM_CORES, n_pad, d), _f32)]
    scratch = [
        pltpu.VMEM((IB, CHUNK), _i32),
        pltpu.VMEM((IB, CHUNK), _i32),
    ]
    scratch += [pltpu.VMEM((CHUNK, d), _f32) for _ in range(NBUF)]
    scratch += [pltpu.SemaphoreType.DMA for _ in range(2 * NBUF)]
    scratch.append(pltpu.VMEM_SHARED((n_pad, d), _f32))
    if with_counts:
        out_type.append(jax.ShapeDtypeStruct((NUM_CORES, n_pad, CNT_W), _f32))
        scratch += [
            pltpu.VMEM((CHUNK, CNT_W), _f32),
            pltpu.VMEM_SHARED((n_pad, CNT_W), _f32),
        ]

    @functools.partial(
        pl.kernel, out_type=tuple(out_type), mesh=mesh,
        scratch_types=scratch,
        compiler_params=pltpu.CompilerParams(use_tc_tiling_on_sc=False))
    def body(table_hbm, src_hbm, dst_hbm, zrow_hbm, *rest):
        if with_counts:
            (zcnt_hbm, ones_hbm, out_sum, out_cnt, src_v, dst_v,
             *rest2) = rest
            rows = rest2[:NBUF]
            gsem = rest2[NBUF:2 * NBUF]
            ssem = rest2[2 * NBUF:3 * NBUF]
            acc_sh, ones_v, cnt_sh = rest2[3 * NBUF:]
        else:
            out_sum, src_v, dst_v, *rest2 = rest
            rows = rest2[:NBUF]
            gsem = rest2[NBUF:2 * NBUF]
            ssem = rest2[2 * NBUF:3 * NBUF]
            (acc_sh,) = rest2[3 * NBUF:]

        cid = lax.axis_index("c")
        sid = lax.axis_index("s")

        pltpu.sync_copy(zrow_hbm, acc_sh.at[pl.ds(sid * rpt, rpt)])
        if with_counts:
            pltpu.sync_copy(zcnt_hbm, cnt_sh.at[pl.ds(sid * rpt, rpt)])
            pltpu.sync_copy(ones_hbm, ones_v)

        plsc.subcore_barrier()

        def drain_scatter(j):
            pltpu.make_async_copy(
                rows[j], acc_sh.at[dst_v.at[j]], ssem[j]).wait()

        def block_body(b, carry):
            @pl.when(b > 0)
            def _():
                for j in range(NBUF):
                    drain_scatter(j)

            base = sid * chunks + b * IB
            pltpu.sync_copy(src_hbm.at[cid, pl.ds(base, IB)], src_v)
            pltpu.sync_copy(dst_hbm.at[cid, pl.ds(base, IB)], dst_v)

            def group_body(g, carry2):
                @pl.when(g > 0)
                def _():
                    for j in range(NBUF):
                        drain_scatter(j)
                for j in range(NBUF):
                    pltpu.async_copy(
                        table_hbm.at[src_v.at[g * NBUF + j]], rows[j],
                        gsem[j])
                if with_counts:
                    for j in range(NBUF):
                        pltpu.sync_copy(
                            ones_v, cnt_sh.at[dst_v.at[g * NBUF + j]],
                            add=True)
                for j in range(NBUF):
                    c = g * NBUF + j
                    pltpu.make_async_copy(
                        table_hbm.at[src_v.at[c]], rows[j], gsem[j]).wait()
                    pltpu.async_copy(
                        rows[j], acc_sh.at[dst_v.at[c]], ssem[j], add=True)
                return carry2

            return lax.fori_loop(0, groups, group_body, carry)

        lax.fori_loop(0, nblocks, block_body, 0)
        for j in range(NBUF):
            drain_scatter(j)
        plsc.subcore_barrier()

        sl = pl.ds(sid * rpt, rpt)
        pltpu.sync_copy(acc_sh.at[sl], out_sum.at[cid, sl])
        if with_counts:
            pltpu.sync_copy(cnt_sh.at[sl], out_cnt.at[cid, sl])

    return body


def _pad_edges(edge_index, e_pad, dummy_dst):
    e = edge_index.shape[1]
    src = jnp.concatenate(
        [edge_index[0], jnp.zeros((e_pad - e,), _i32)]).reshape(-1, CHUNK)
    dst = jnp.concatenate(
        [edge_index[1], jnp.full((e_pad - e,), dummy_dst, _i32)]
    ).reshape(-1, CHUNK)
    return src, dst


def _row_spec(r, cols):
    return pl.BlockSpec((r, cols), lambda i: (i, 0))


def _pick_spec(which, r, cols):
    return pl.BlockSpec((1, r, cols), lambda i, w=which: (w, i, 0))


def _full_spec(shape):
    nd = len(shape)
    return pl.BlockSpec(shape, lambda i: (0,) * nd)


def _recip(cnt_ref):
    return 1.0 / jnp.maximum(cnt_ref[0][:, :1], 1.0)


def _base_dense_body(sum_p, sum_n, cnt_p, cnt_n, x, wp, wn, bp, bn, out):
    agg_p = sum_p[0] * _recip(cnt_p)
    agg_n = sum_n[0] * _recip(cnt_n)
    xb = x[...]
    fp = jnp.concatenate([agg_p, xb], axis=1)
    fn = jnp.concatenate([agg_n, xb], axis=1)
    hp = jnp.tanh(jnp.dot(fp, wp[...], preferred_element_type=_f32) + bp[...])
    hn = jnp.tanh(jnp.dot(fn, wn[...], preferred_element_type=_f32) + bn[...])
    out[...] = jnp.concatenate([hp, hn], axis=1)


def _deep_dense_body(s_p, s_n, cnt_p, cnt_n, hcat, wp, wn, bp, bn, out):
    rp = _recip(cnt_p)
    rn = _recip(cnt_n)
    sp = s_p[0]
    sn = s_n[0]
    hb = hcat[...]
    hp, hn = hb[:, :32], hb[:, 32:]
    feat = jnp.concatenate([
        sp[:, :32] * rp,
        sn[:, 32:] * rn,
        sp[:, 32:] * rp,
        sn[:, :32] * rn,
        hp, hn, 0.5 * (hp + hn),
    ], axis=1)
    np_ = jnp.tanh(jnp.dot(feat, wp[...], preferred_element_type=_f32) + bp[...])
    nn_ = jnp.tanh(jnp.dot(feat, wn[...], preferred_element_type=_f32) + bn[...])
    out[...] = jnp.concatenate([np_, nn_], axis=1)


def _lstm_body(emb, w_ih_t, w_hh_t, bias, out):
    eb = emb[...]
    cells = w_ih_t.shape[0]
    h = jnp.zeros((eb.shape[0], 64), _f32)
    c = jnp.zeros((eb.shape[0], 64), _f32)
    for i in range(cells):
        gates = (jnp.dot(eb, w_ih_t[i], preferred_element_type=_f32)
                 + jnp.dot(h, w_hh_t[i], preferred_element_type=_f32)
                 + bias[i])
        ig = jax.nn.sigmoid(gates[:, :64])
        fg = jax.nn.sigmoid(gates[:, 64:128])
        gg = jnp.tanh(gates[:, 128:192])
        og = jax.nn.sigmoid(gates[:, 192:256])
        c = fg * c + ig * gg
        h = og * jnp.tanh(c)
    out[...] = h


def kernel(x, pos_edge_index, neg_edge_index, W_pos_base, b_pos_base,
           W_neg_base, b_neg_base, W_pos_deep, b_pos_deep, W_neg_deep,
           b_neg_deep, W_ih, W_hh, b_ih, b_hh):
    n, d_feat = x.shape
    e = pos_edge_index.shape[1]
    hid = W_pos_base.shape[1]
    dcat = 2 * hid

    n_pad = ((n + R_BLOCK - 1) // R_BLOCK) * R_BLOCK
    grid = n_pad // R_BLOCK
    epc = NUM_TILES * CHUNK
    chunks = (((e + epc - 1) // epc + IB - 1) // IB) * IB
    e_pad = chunks * epc

    src_p, dst_p = _pad_edges(pos_edge_index, e_pad, n_pad - 1)
    src_n, dst_n = _pad_edges(neg_edge_index, e_pad, n_pad - 1)
    src_all = jnp.stack([src_p, src_n])
    dst_all = jnp.stack([dst_p, dst_n])

    rpt = n_pad // NUM_TILES
    zrow_base = jnp.zeros((rpt, d_feat), _f32)
    zrow_deep = jnp.zeros((rpt, dcat), _f32)
    zcnt = jnp.zeros((rpt, CNT_W), _f32)
    ones = jnp.ones((CHUNK, CNT_W), _f32)

    x_pad = jnp.pad(x, ((0, n_pad - n), (0, 0)))

    seg_base = _make_seg_sum(n_pad, d_feat, chunks, True)
    sums, cnts = seg_base(x, src_all, dst_all, zrow_base, zcnt, ones)

    bp = b_pos_base.reshape(1, hid)
    bn = b_neg_base.reshape(1, hid)
    hcat = pl.pallas_call(
        _base_dense_body,
        grid=(grid,),
        in_specs=[
            _pick_spec(0, R_BLOCK, d_feat), _pick_spec(1, R_BLOCK, d_feat),
            _pick_spec(0, R_BLOCK, CNT_W), _pick_spec(1, R_BLOCK, CNT_W),
            _row_spec(R_BLOCK, d_feat),
            _full_spec(W_pos_base.shape), _full_spec(W_neg_base.shape),
            _full_spec(bp.shape), _full_spec(bn.shape),
        ],
        out_specs=_row_spec(R_BLOCK, dcat),
        out_shape=jax.ShapeDtypeStruct((n_pad, dcat), _f32),
    )(sums, sums, cnts, cnts, x_pad, W_pos_base, W_neg_base, bp, bn)

    seg_deep = _make_seg_sum(n_pad, dcat, chunks, False)
    for i in range(W_pos_deep.shape[0]):
        (dsums,) = seg_deep(hcat, src_all, dst_all, zrow_deep)
        bpi = b_pos_deep[i].reshape(1, hid)
        bni = b_neg_deep[i].reshape(1, hid)
        hcat = pl.pallas_call(
            _deep_dense_body,
            grid=(grid,),
            in_specs=[
                _pick_spec(0, R_BLOCK, dcat), _pick_spec(1, R_BLOCK, dcat),
                _pick_spec(0, R_BLOCK, CNT_W), _pick_spec(1, R_BLOCK, CNT_W),
                _row_spec(R_BLOCK, dcat),
                _full_spec(W_pos_deep[i].shape),
                _full_spec(W_neg_deep[i].shape),
                _full_spec(bpi.shape), _full_spec(bni.shape),
            ],
            out_specs=_row_spec(R_BLOCK, dcat),
            out_shape=jax.ShapeDtypeStruct((n_pad, dcat), _f32),
        )(dsums, dsums, cnts, cnts, hcat, W_pos_deep[i], W_neg_deep[i],
          bpi, bni)

    w_ih_t = W_ih.transpose(0, 2, 1)
    w_hh_t = W_hh.transpose(0, 2, 1)
    bias = (b_ih + b_hh).reshape(W_ih.shape[0], 1, W_ih.shape[1])
    hx = pl.pallas_call(
        _lstm_body,
        grid=(grid,),
        in_specs=[
            _row_spec(R_BLOCK, dcat),
            _full_spec(w_ih_t.shape), _full_spec(w_hh_t.shape),
            _full_spec(bias.shape),
        ],
        out_specs=_row_spec(R_BLOCK, 64),
        out_shape=jax.ShapeDtypeStruct((n_pad, 64), _f32),
    )(hcat, w_ih_t, w_hh_t, bias)

    return hx[:n]


# speedup vs baseline: 6.7343x; 1.0126x over previous
import functools

import jax
import jax.numpy as jnp
from jax import lax
from jax.experimental import pallas as pl
from jax.experimental.pallas import tpu as pltpu
from jax.experimental.pallas import tpu_sc as plsc

NUM_CORES = 2
NUM_TILES = 16
CHUNK = 64
NBUF = 4
IB = 16
CNT_W = 16
R_BLOCK = 1024

_f32 = jnp.float32
_i32 = jnp.int32


@functools.cache
def _make_seg_sum(n_pad: int, d: int, ck: int, chunks: int,
                  with_counts: bool):
    rpt = n_pad // NUM_TILES
    nblocks = chunks // IB
    groups = IB // NBUF

    mesh = plsc.VectorSubcoreMesh(
        core_axis_name="c", subcore_axis_name="s",
        num_cores=NUM_CORES, num_subcores=NUM_TILES)

    out_type = [jax.ShapeDtypeStruct((NUM_CORES, n_pad, d), _f32)]
    scratch = [
        pltpu.VMEM((IB, ck), _i32),
        pltpu.VMEM((IB, ck), _i32),
    ]
    scratch += [pltpu.VMEM((ck, d), _f32) for _ in range(NBUF)]
    scratch += [pltpu.SemaphoreType.DMA for _ in range(2 * NBUF)]
    scratch.append(pltpu.VMEM_SHARED((n_pad, d), _f32))
    if with_counts:
        out_type.append(jax.ShapeDtypeStruct((NUM_CORES, n_pad, CNT_W), _f32))
        scratch += [
            pltpu.SemaphoreType.DMA,
            pltpu.VMEM((ck, CNT_W), _f32),
            pltpu.VMEM_SHARED((n_pad, CNT_W), _f32),
        ]

    @functools.partial(
        pl.kernel, out_type=tuple(out_type), mesh=mesh,
        scratch_types=scratch,
        compiler_params=pltpu.CompilerParams(use_tc_tiling_on_sc=False))
    def body(table_hbm, src_hbm, dst_hbm, zrow_hbm, *rest):
        if with_counts:
            (zcnt_hbm, ones_hbm, out_sum, out_cnt, src_v, dst_v,
             *rest2) = rest
        else:
            out_sum, src_v, dst_v, *rest2 = rest
        rows = rest2[:NBUF]
        gsem = rest2[NBUF:2 * NBUF]
        ssem = rest2[2 * NBUF:3 * NBUF]
        if with_counts:
            csem, ones_v, cnt_sh = rest2[3 * NBUF + 1:]
            acc_sh = rest2[3 * NBUF]
        else:
            (acc_sh,) = rest2[3 * NBUF:]

        cid = lax.axis_index("c")
        sid = lax.axis_index("s")

        pltpu.sync_copy(zrow_hbm, acc_sh.at[pl.ds(sid * rpt, rpt)])
        if with_counts:
            pltpu.sync_copy(zcnt_hbm, cnt_sh.at[pl.ds(sid * rpt, rpt)])
            pltpu.sync_copy(ones_hbm, ones_v)

        plsc.subcore_barrier()

        def drain_scatter(j):
            pltpu.make_async_copy(
                rows[j], acc_sh.at[dst_v.at[j]], ssem[j]).wait()

        def block_body(b, carry):
            @pl.when(b > 0)
            def _():
                for j in range(NBUF):
                    drain_scatter(j)
                if with_counts:
                    def cdrain(k, cc):
                        pltpu.make_async_copy(
                            ones_v, cnt_sh.at[dst_v.at[0]], csem).wait()
                        return cc
                    lax.fori_loop(0, IB, cdrain, 0)

            base = sid * chunks + b * IB
            pltpu.sync_copy(src_hbm.at[cid, pl.ds(base, IB)], src_v)
            pltpu.sync_copy(dst_hbm.at[cid, pl.ds(base, IB)], dst_v)

            def group_body(g, carry2):
                @pl.when(g > 0)
                def _():
                    for j in range(NBUF):
                        drain_scatter(j)
                for j in range(NBUF):
                    pltpu.async_copy(
                        table_hbm.at[src_v.at[g * NBUF + j]], rows[j],
                        gsem[j])
                if with_counts:
                    for j in range(NBUF):
                        pltpu.async_copy(
                            ones_v, cnt_sh.at[dst_v.at[g * NBUF + j]],
                            csem, add=True)
                for j in range(NBUF):
                    c = g * NBUF + j
                    pltpu.make_async_copy(
                        table_hbm.at[src_v.at[c]], rows[j], gsem[j]).wait()
                    pltpu.async_copy(
                        rows[j], acc_sh.at[dst_v.at[c]], ssem[j], add=True)
                return carry2

            return lax.fori_loop(0, groups, group_body, carry)

        lax.fori_loop(0, nblocks, block_body, 0)
        for j in range(NBUF):
            drain_scatter(j)
        if with_counts:
            def cdrain_end(k, cc):
                pltpu.make_async_copy(
                    ones_v, cnt_sh.at[dst_v.at[0]], csem).wait()
                return cc
            lax.fori_loop(0, IB, cdrain_end, 0)
        plsc.subcore_barrier()

        sl = pl.ds(sid * rpt, rpt)
        pltpu.sync_copy(acc_sh.at[sl], out_sum.at[cid, sl])
        if with_counts:
            pltpu.sync_copy(cnt_sh.at[sl], out_cnt.at[cid, sl])

    return body


def _pad_edges(edge_index, e_pad, dummy_dst, ck):
    e = edge_index.shape[1]
    src = jnp.concatenate(
        [edge_index[0], jnp.zeros((e_pad - e,), _i32)]).reshape(-1, ck)
    dst = jnp.concatenate(
        [edge_index[1], jnp.full((e_pad - e,), dummy_dst, _i32)]
    ).reshape(-1, ck)
    return src, dst


def _row_spec(r, cols):
    return pl.BlockSpec((r, cols), lambda i: (i, 0))


def _pick_spec(which, r, cols):
    return pl.BlockSpec((1, r, cols), lambda i, w=which: (w, i, 0))


def _full_spec(shape):
    nd = len(shape)
    return pl.BlockSpec(shape, lambda i: (0,) * nd)


def _recip(cnt_ref):
    return 1.0 / jnp.maximum(cnt_ref[0][:, :1], 1.0)


def _base_dense_body(sum_p, sum_n, cnt_p, cnt_n, x, wp, wn, bp, bn, out):
    agg_p = sum_p[0] * _recip(cnt_p)
    agg_n = sum_n[0] * _recip(cnt_n)
    xb = x[...]
    fp = jnp.concatenate([agg_p, xb], axis=1)
    fn = jnp.concatenate([agg_n, xb], axis=1)
    hp = jnp.tanh(jnp.dot(fp, wp[...], preferred_element_type=_f32) + bp[...])
    hn = jnp.tanh(jnp.dot(fn, wn[...], preferred_element_type=_f32) + bn[...])
    out[...] = jnp.concatenate([hp, hn], axis=1)


def _deep_dense_body(s_p, s_n, cnt_p, cnt_n, hcat, wp, wn, bp, bn, out):
    rp = _recip(cnt_p)
    rn = _recip(cnt_n)
    sp = s_p[0]
    sn = s_n[0]
    hb = hcat[...]
    hp, hn = hb[:, :32], hb[:, 32:]
    feat = jnp.concatenate([
        sp[:, :32] * rp,
        sn[:, 32:] * rn,
        sp[:, 32:] * rp,
        sn[:, :32] * rn,
        hp, hn, 0.5 * (hp + hn),
    ], axis=1)
    np_ = jnp.tanh(jnp.dot(feat, wp[...], preferred_element_type=_f32) + bp[...])
    nn_ = jnp.tanh(jnp.dot(feat, wn[...], preferred_element_type=_f32) + bn[...])
    out[...] = jnp.concatenate([np_, nn_], axis=1)


def _lstm_body(emb, w_ih_t, w_hh_t, bias, out):
    eb = emb[...]
    cells = w_ih_t.shape[0]
    h = jnp.zeros((eb.shape[0], 64), _f32)
    c = jnp.zeros((eb.shape[0], 64), _f32)
    for i in range(cells):
        gates = (jnp.dot(eb, w_ih_t[i], preferred_element_type=_f32)
                 + jnp.dot(h, w_hh_t[i], preferred_element_type=_f32)
                 + bias[i])
        ig = jax.nn.sigmoid(gates[:, :64])
        fg = jax.nn.sigmoid(gates[:, 64:128])
        gg = jnp.tanh(gates[:, 128:192])
        og = jax.nn.sigmoid(gates[:, 192:256])
        c = fg * c + ig * gg
        h = og * jnp.tanh(c)
    out[...] = h


def kernel(x, pos_edge_index, neg_edge_index, W_pos_base, b_pos_base,
           W_neg_base, b_neg_base, W_pos_deep, b_pos_deep, W_neg_deep,
           b_neg_deep, W_ih, W_hh, b_ih, b_hh):
    n, d_feat = x.shape
    e = pos_edge_index.shape[1]
    hid = W_pos_base.shape[1]
    dcat = 2 * hid

    n_pad = ((n + R_BLOCK - 1) // R_BLOCK) * R_BLOCK
    grid = n_pad // R_BLOCK
    ck_base, ck_deep = 64, 128
    quantum = NUM_TILES * IB * ck_deep
    e_pad = ((e + quantum - 1) // quantum) * quantum
    chunks_base = e_pad // (NUM_TILES * ck_base)
    chunks_deep = e_pad // (NUM_TILES * ck_deep)

    src_pb, dst_pb = _pad_edges(pos_edge_index, e_pad, n_pad - 1, ck_base)
    src_nb, dst_nb = _pad_edges(neg_edge_index, e_pad, n_pad - 1, ck_base)
    src_b = jnp.stack([src_pb, src_nb])
    dst_b = jnp.stack([dst_pb, dst_nb])
    src_d = src_b.reshape(NUM_CORES, -1, ck_deep)
    dst_d = dst_b.reshape(NUM_CORES, -1, ck_deep)

    rpt = n_pad // NUM_TILES
    zrow_base = jnp.zeros((rpt, d_feat), _f32)
    zrow_deep = jnp.zeros((rpt, dcat), _f32)
    zcnt = jnp.zeros((rpt, CNT_W), _f32)
    ones = jnp.ones((ck_base, CNT_W), _f32)

    x_pad = jnp.pad(x, ((0, n_pad - n), (0, 0)))

    seg_base = _make_seg_sum(n_pad, d_feat, ck_base, chunks_base, True)
    sums, cnts = seg_base(x, src_b, dst_b, zrow_base, zcnt, ones)

    bp = b_pos_base.reshape(1, hid)
    bn = b_neg_base.reshape(1, hid)
    hcat = pl.pallas_call(
        _base_dense_body,
        grid=(grid,),
        in_specs=[
            _pick_spec(0, R_BLOCK, d_feat), _pick_spec(1, R_BLOCK, d_feat),
            _pick_spec(0, R_BLOCK, CNT_W), _pick_spec(1, R_BLOCK, CNT_W),
            _row_spec(R_BLOCK, d_feat),
            _full_spec(W_pos_base.shape), _full_spec(W_neg_base.shape),
            _full_spec(bp.shape), _full_spec(bn.shape),
        ],
        out_specs=_row_spec(R_BLOCK, dcat),
        out_shape=jax.ShapeDtypeStruct((n_pad, dcat), _f32),
    )(sums, sums, cnts, cnts, x_pad, W_pos_base, W_neg_base, bp, bn)

    seg_deep = _make_seg_sum(n_pad, dcat, ck_deep, chunks_deep, False)
    for i in range(W_pos_deep.shape[0]):
        (dsums,) = seg_deep(hcat, src_d, dst_d, zrow_deep)
        bpi = b_pos_deep[i].reshape(1, hid)
        bni = b_neg_deep[i].reshape(1, hid)
        hcat = pl.pallas_call(
            _deep_dense_body,
            grid=(grid,),
            in_specs=[
                _pick_spec(0, R_BLOCK, dcat), _pick_spec(1, R_BLOCK, dcat),
                _pick_spec(0, R_BLOCK, CNT_W), _pick_spec(1, R_BLOCK, CNT_W),
                _row_spec(R_BLOCK, dcat),
                _full_spec(W_pos_deep[i].shape),
                _full_spec(W_neg_deep[i].shape),
                _full_spec(bpi.shape), _full_spec(bni.shape),
            ],
            out_specs=_row_spec(R_BLOCK, dcat),
            out_shape=jax.ShapeDtypeStruct((n_pad, dcat), _f32),
        )(dsums, dsums, cnts, cnts, hcat, W_pos_deep[i], W_neg_deep[i],
          bpi, bni)

    w_ih_t = W_ih.transpose(0, 2, 1)
    w_hh_t = W_hh.transpose(0, 2, 1)
    bias = (b_ih + b_hh).reshape(W_ih.shape[0], 1, W_ih.shape[1])
    hx = pl.pallas_call(
        _lstm_body,
        grid=(grid,),
        in_specs=[
            _row_spec(R_BLOCK, dcat),
            _full_spec(w_ih_t.shape), _full_spec(w_hh_t.shape),
            _full_spec(bias.shape),
        ],
        out_specs=_row_spec(R_BLOCK, 64),
        out_shape=jax.ShapeDtypeStruct((n_pad, 64), _f32),
    )(hcat, w_ih_t, w_hh_t, bias)

    return hx[:n]


# speedup vs baseline: 9.4393x; 1.4017x over previous
import functools

import jax
import jax.numpy as jnp
from jax import lax
from jax.experimental import pallas as pl
from jax.experimental.pallas import tpu as pltpu
from jax.experimental.pallas import tpu_sc as plsc

NUM_CORES = 2
NUM_TILES = 16
NBUF = 4
IB = 16
R_BLOCK = 1024

_f32 = jnp.float32
_i32 = jnp.int32


@functools.cache
def _make_seg_sum(n_pad: int, d: int, ck: int, chunks: int):
    rpt = n_pad // NUM_TILES
    nblocks = chunks // IB
    groups = IB // NBUF

    mesh = plsc.VectorSubcoreMesh(
        core_axis_name="c", subcore_axis_name="s",
        num_cores=NUM_CORES, num_subcores=NUM_TILES)

    scratch = [
        pltpu.VMEM((IB, ck), _i32),
        pltpu.VMEM((IB, ck), _i32),
    ]
    scratch += [pltpu.VMEM((ck, d), _f32) for _ in range(NBUF)]
    scratch += [pltpu.SemaphoreType.DMA for _ in range(2 * NBUF)]
    scratch.append(pltpu.VMEM_SHARED((n_pad, d), _f32))

    @functools.partial(
        pl.kernel,
        out_type=jax.ShapeDtypeStruct((NUM_CORES, n_pad, d), _f32),
        mesh=mesh, scratch_types=scratch,
        compiler_params=pltpu.CompilerParams(use_tc_tiling_on_sc=False))
    def body(table_hbm, src_hbm, dst_hbm, zrow_hbm, out_sum,
             src_v, dst_v, *rest):
        rows = rest[:NBUF]
        gsem = rest[NBUF:2 * NBUF]
        ssem = rest[2 * NBUF:3 * NBUF]
        (acc_sh,) = rest[3 * NBUF:]

        cid = lax.axis_index("c")
        sid = lax.axis_index("s")

        pltpu.sync_copy(zrow_hbm, acc_sh.at[pl.ds(sid * rpt, rpt)])
        plsc.subcore_barrier()

        def drain_scatter(j):
            pltpu.make_async_copy(
                rows[j], acc_sh.at[dst_v.at[j]], ssem[j]).wait()

        def block_body(b, carry):
            @pl.when(b > 0)
            def _():
                for j in range(NBUF):
                    drain_scatter(j)

            base = sid * chunks + b * IB
            pltpu.sync_copy(src_hbm.at[cid, pl.ds(base, IB)], src_v)
            pltpu.sync_copy(dst_hbm.at[cid, pl.ds(base, IB)], dst_v)

            def group_body(g, carry2):
                @pl.when(g > 0)
                def _():
                    for j in range(NBUF):
                        drain_scatter(j)
                for j in range(NBUF):
                    pltpu.async_copy(
                        table_hbm.at[src_v.at[g * NBUF + j]], rows[j],
                        gsem[j])
                for j in range(NBUF):
                    c = g * NBUF + j
                    pltpu.make_async_copy(
                        table_hbm.at[src_v.at[c]], rows[j], gsem[j]).wait()
                    pltpu.async_copy(
                        rows[j], acc_sh.at[dst_v.at[c]], ssem[j], add=True)
                return carry2

            return lax.fori_loop(0, groups, group_body, carry)

        lax.fori_loop(0, nblocks, block_body, 0)
        for j in range(NBUF):
            drain_scatter(j)
        plsc.subcore_barrier()

        sl = pl.ds(sid * rpt, rpt)
        pltpu.sync_copy(acc_sh.at[sl], out_sum.at[cid, sl])

    return body


def _pad_edges(edge_index, e_pad, dummy_dst, ck):
    e = edge_index.shape[1]
    src = jnp.concatenate(
        [edge_index[0], jnp.zeros((e_pad - e,), _i32)]).reshape(-1, ck)
    dst = jnp.concatenate(
        [edge_index[1], jnp.full((e_pad - e,), dummy_dst, _i32)]
    ).reshape(-1, ck)
    return src, dst


def _row_spec(r, cols):
    return pl.BlockSpec((r, cols), lambda i: (i, 0))


def _pick_spec(which, r, cols):
    return pl.BlockSpec((1, r, cols), lambda i, w=which: (w, i, 0))


def _full_spec(shape):
    nd = len(shape)
    return pl.BlockSpec(shape, lambda i: (0,) * nd)


def _recip(sums_ref):
    return 1.0 / jnp.maximum(sums_ref[0][:, 32:33], 1.0)


def _proj_body(x, wpa, wna, yp, yn):
    xb = x[...]
    r = xb.shape[0]
    one = jnp.ones((r, 1), _f32)
    pad = jnp.zeros((r, 15), _f32)
    yp[...] = jnp.concatenate(
        [jnp.dot(xb, wpa[...], preferred_element_type=_f32), one, pad], 1)
    yn[...] = jnp.concatenate(
        [jnp.dot(xb, wna[...], preferred_element_type=_f32), one, pad], 1)


def _base_dense_body(sum_p, sum_n, x, wpx, wnx, bp, bn, out):
    agg_p = sum_p[0][:, :32] * _recip(sum_p)
    agg_n = sum_n[0][:, :32] * _recip(sum_n)
    xb = x[...]
    hp = jnp.tanh(
        agg_p + jnp.dot(xb, wpx[...], preferred_element_type=_f32) + bp[...])
    hn = jnp.tanh(
        agg_n + jnp.dot(xb, wnx[...], preferred_element_type=_f32) + bn[...])
    out[...] = jnp.concatenate([hp, hn], axis=1)


def _deep_dense_body(s_p, s_n, cnt_p, cnt_n, hcat, wp, wn, bp, bn, out):
    rp = _recip(cnt_p)
    rn = _recip(cnt_n)
    sp = s_p[0]
    sn = s_n[0]
    hb = hcat[...]
    hp, hn = hb[:, :32], hb[:, 32:]
    feat = jnp.concatenate([
        sp[:, :32] * rp,
        sn[:, 32:] * rn,
        sp[:, 32:] * rp,
        sn[:, :32] * rn,
        hp, hn, 0.5 * (hp + hn),
    ], axis=1)
    np_ = jnp.tanh(jnp.dot(feat, wp[...], preferred_element_type=_f32) + bp[...])
    nn_ = jnp.tanh(jnp.dot(feat, wn[...], preferred_element_type=_f32) + bn[...])
    out[...] = jnp.concatenate([np_, nn_], axis=1)


def _lstm_body(emb, w_ih_t, w_hh_t, bias, out):
    eb = emb[...]
    cells = w_ih_t.shape[0]
    h = jnp.zeros((eb.shape[0], 64), _f32)
    c = jnp.zeros((eb.shape[0], 64), _f32)
    for i in range(cells):
        gates = (jnp.dot(eb, w_ih_t[i], preferred_element_type=_f32)
                 + jnp.dot(h, w_hh_t[i], preferred_element_type=_f32)
                 + bias[i])
        ig = jax.nn.sigmoid(gates[:, :64])
        fg = jax.nn.sigmoid(gates[:, 64:128])
        gg = jnp.tanh(gates[:, 128:192])
        og = jax.nn.sigmoid(gates[:, 192:256])
        c = fg * c + ig * gg
        h = og * jnp.tanh(c)
    out[...] = h


def kernel(x, pos_edge_index, neg_edge_index, W_pos_base, b_pos_base,
           W_neg_base, b_neg_base, W_pos_deep, b_pos_deep, W_neg_deep,
           b_neg_deep, W_ih, W_hh, b_ih, b_hh):
    n, d_feat = x.shape
    e = pos_edge_index.shape[1]
    hid = W_pos_base.shape[1]
    dcat = 2 * hid

    n_pad = ((n + R_BLOCK - 1) // R_BLOCK) * R_BLOCK
    grid = n_pad // R_BLOCK
    ck = 128
    dproj = hid + 16
    quantum = NUM_TILES * IB * ck
    e_pad = ((e + quantum - 1) // quantum) * quantum
    chunks = e_pad // (NUM_TILES * ck)

    src_p, dst_p = _pad_edges(pos_edge_index, e_pad, n_pad - 1, ck)
    src_n, dst_n = _pad_edges(neg_edge_index, e_pad, n_pad - 1, ck)
    src_d = jnp.stack([src_p, src_n])
    dst_d = jnp.stack([dst_p, dst_n])
    src_b = jnp.stack([src_p, src_n + n_pad])

    rpt = n_pad // NUM_TILES
    zrow_base = jnp.zeros((rpt, dproj), _f32)
    zrow_deep = jnp.zeros((rpt, dcat), _f32)

    x_pad = jnp.pad(x, ((0, n_pad - n), (0, 0)))

    yp, yn = pl.pallas_call(
        _proj_body,
        grid=(grid,),
        in_specs=[
            _row_spec(R_BLOCK, d_feat),
            _full_spec((d_feat, hid)), _full_spec((d_feat, hid)),
        ],
        out_specs=[_row_spec(R_BLOCK, dproj), _row_spec(R_BLOCK, dproj)],
        out_shape=[jax.ShapeDtypeStruct((n_pad, dproj), _f32)] * 2,
    )(x_pad, W_pos_base[:d_feat], W_neg_base[:d_feat])
    ytab = jnp.concatenate([yp, yn], axis=0)

    seg_base = _make_seg_sum(n_pad, dproj, ck, chunks)
    sums = seg_base(ytab, src_b, dst_d, zrow_base)

    bp = b_pos_base.reshape(1, hid)
    bn = b_neg_base.reshape(1, hid)
    hcat = pl.pallas_call(
        _base_dense_body,
        grid=(grid,),
        in_specs=[
            _pick_spec(0, R_BLOCK, dproj), _pick_spec(1, R_BLOCK, dproj),
            _row_spec(R_BLOCK, d_feat),
            _full_spec((d_feat, hid)), _full_spec((d_feat, hid)),
            _full_spec(bp.shape), _full_spec(bn.shape),
        ],
        out_specs=_row_spec(R_BLOCK, dcat),
        out_shape=jax.ShapeDtypeStruct((n_pad, dcat), _f32),
    )(sums, sums, x_pad, W_pos_base[d_feat:], W_neg_base[d_feat:], bp, bn)

    seg_deep = _make_seg_sum(n_pad, dcat, ck, chunks)
    for i in range(W_pos_deep.shape[0]):
        dsums = seg_deep(hcat, src_d, dst_d, zrow_deep)
        bpi = b_pos_deep[i].reshape(1, hid)
        bni = b_neg_deep[i].reshape(1, hid)
        hcat = pl.pallas_call(
            _deep_dense_body,
            grid=(grid,),
            in_specs=[
                _pick_spec(0, R_BLOCK, dcat), _pick_spec(1, R_BLOCK, dcat),
                _pick_spec(0, R_BLOCK, dproj), _pick_spec(1, R_BLOCK, dproj),
                _row_spec(R_BLOCK, dcat),
                _full_spec(W_pos_deep[i].shape),
                _full_spec(W_neg_deep[i].shape),
                _full_spec(bpi.shape), _full_spec(bni.shape),
            ],
            out_specs=_row_spec(R_BLOCK, dcat),
            out_shape=jax.ShapeDtypeStruct((n_pad, dcat), _f32),
        )(dsums, dsums, sums, sums, hcat, W_pos_deep[i], W_neg_deep[i],
          bpi, bni)

    w_ih_t = W_ih.transpose(0, 2, 1)
    w_hh_t = W_hh.transpose(0, 2, 1)
    bias = (b_ih + b_hh).reshape(W_ih.shape[0], 1, W_ih.shape[1])
    hx = pl.pallas_call(
        _lstm_body,
        grid=(grid,),
        in_specs=[
            _row_spec(R_BLOCK, dcat),
            _full_spec(w_ih_t.shape), _full_spec(w_hh_t.shape),
            _full_spec(bias.shape),
        ],
        out_specs=_row_spec(R_BLOCK, 64),
        out_shape=jax.ShapeDtypeStruct((n_pad, 64), _f32),
    )(hcat, w_ih_t, w_hh_t, bias)

    return hx[:n]


# speedup vs baseline: 9.5238x; 1.0089x over previous
import functools

import jax
import jax.numpy as jnp
from jax import lax
from jax.experimental import pallas as pl
from jax.experimental.pallas import tpu as pltpu
from jax.experimental.pallas import tpu_sc as plsc

NUM_CORES = 2
NUM_TILES = 16
NBUF = 4
IB = 16
R_BLOCK = 1024

_f32 = jnp.float32
_i32 = jnp.int32


@functools.cache
def _make_seg_sum(n_pad: int, d: int, ck: int, chunks: int):
    rpt = n_pad // NUM_TILES
    nblocks = chunks // IB
    groups = IB // NBUF

    mesh = plsc.VectorSubcoreMesh(
        core_axis_name="c", subcore_axis_name="s",
        num_cores=NUM_CORES, num_subcores=NUM_TILES)

    scratch = [
        pltpu.VMEM((IB, ck), _i32),
        pltpu.VMEM((IB, ck), _i32),
    ]
    scratch += [pltpu.VMEM((ck, d), _f32) for _ in range(NBUF)]
    scratch += [pltpu.SemaphoreType.DMA for _ in range(2 * NBUF)]
    scratch.append(pltpu.VMEM_SHARED((n_pad, d), _f32))

    @functools.partial(
        pl.kernel,
        out_type=jax.ShapeDtypeStruct((NUM_CORES, n_pad, d), _f32),
        mesh=mesh, scratch_types=scratch,
        compiler_params=pltpu.CompilerParams(use_tc_tiling_on_sc=False))
    def body(table_hbm, src_hbm, dst_hbm, zrow_hbm, out_sum,
             src_v, dst_v, *rest):
        rows = rest[:NBUF]
        gsem = rest[NBUF:2 * NBUF]
        ssem = rest[2 * NBUF:3 * NBUF]
        (acc_sh,) = rest[3 * NBUF:]

        cid = lax.axis_index("c")
        sid = lax.axis_index("s")

        pltpu.sync_copy(zrow_hbm, acc_sh.at[pl.ds(sid * rpt, rpt)])
        plsc.subcore_barrier()

        def drain_scatter(j):
            pltpu.make_async_copy(
                rows[j], acc_sh.at[dst_v.at[j]], ssem[j]).wait()

        def block_body(b, carry):
            @pl.when(b > 0)
            def _():
                for j in range(NBUF):
                    drain_scatter(j)

            base = sid * chunks + b * IB
            pltpu.sync_copy(src_hbm.at[cid, pl.ds(base, IB)], src_v)
            pltpu.sync_copy(dst_hbm.at[cid, pl.ds(base, IB)], dst_v)

            def group_body(g, carry2):
                @pl.when(g > 0)
                def _():
                    for j in range(NBUF):
                        drain_scatter(j)
                for j in range(NBUF):
                    pltpu.async_copy(
                        table_hbm.at[src_v.at[g * NBUF + j]], rows[j],
                        gsem[j])
                for j in range(NBUF):
                    c = g * NBUF + j
                    pltpu.make_async_copy(
                        table_hbm.at[src_v.at[c]], rows[j], gsem[j]).wait()
                    pltpu.async_copy(
                        rows[j], acc_sh.at[dst_v.at[c]], ssem[j], add=True)
                return carry2

            return lax.fori_loop(0, groups, group_body, carry)

        lax.fori_loop(0, nblocks, block_body, 0)
        for j in range(NBUF):
            drain_scatter(j)
        plsc.subcore_barrier()

        sl = pl.ds(sid * rpt, rpt)
        pltpu.sync_copy(acc_sh.at[sl], out_sum.at[cid, sl])

    return body


def _pad_edges(edge_index, e_pad, dummy_dst, ck):
    e = edge_index.shape[1]
    src = jnp.concatenate(
        [edge_index[0], jnp.zeros((e_pad - e,), _i32)]).reshape(-1, ck)
    dst = jnp.concatenate(
        [edge_index[1], jnp.full((e_pad - e,), dummy_dst, _i32)]
    ).reshape(-1, ck)
    return src, dst


def _row_spec(r, cols):
    return pl.BlockSpec((r, cols), lambda i: (i, 0))


def _pick_spec(which, r, cols):
    return pl.BlockSpec((1, r, cols), lambda i, w=which: (w, i, 0))


def _full_spec(shape):
    nd = len(shape)
    return pl.BlockSpec(shape, lambda i: (0,) * nd)


def _recip(sums_ref):
    return 1.0 / jnp.maximum(sums_ref[0][:, 32:33], 1.0)


def _proj_body(x, wa, y):
    xb = x[...]
    r = xb.shape[0]
    one = jnp.ones((r, 1), _f32)
    pad = jnp.zeros((r, 15), _f32)
    y[...] = jnp.concatenate(
        [jnp.dot(xb, wa[0], preferred_element_type=_f32), one, pad], 1)


def _base_dense_body(sum_p, sum_n, x, wpx, wnx, bp, bn, out):
    agg_p = sum_p[0][:, :32] * _recip(sum_p)
    agg_n = sum_n[0][:, :32] * _recip(sum_n)
    xb = x[...]
    hp = jnp.tanh(
        agg_p + jnp.dot(xb, wpx[...], preferred_element_type=_f32) + bp[...])
    hn = jnp.tanh(
        agg_n + jnp.dot(xb, wnx[...], preferred_element_type=_f32) + bn[...])
    out[...] = jnp.concatenate([hp, hn], axis=1)


def _deep_emb(s_p, s_n, cnt_p, cnt_n, hcat, wp, wn, bp, bn):
    rp = _recip(cnt_p)
    rn = _recip(cnt_n)
    sp = s_p[0]
    sn = s_n[0]
    hb = hcat[...]
    hp, hn = hb[:, :32], hb[:, 32:]
    feat = jnp.concatenate([
        sp[:, :32] * rp,
        sn[:, 32:] * rn,
        sp[:, 32:] * rp,
        sn[:, :32] * rn,
        hp, hn, 0.5 * (hp + hn),
    ], axis=1)
    np_ = jnp.tanh(jnp.dot(feat, wp[...], preferred_element_type=_f32) + bp[...])
    nn_ = jnp.tanh(jnp.dot(feat, wn[...], preferred_element_type=_f32) + bn[...])
    return jnp.concatenate([np_, nn_], axis=1)


def _deep_dense_body(s_p, s_n, cnt_p, cnt_n, hcat, wp, wn, bp, bn, out):
    out[...] = _deep_emb(s_p, s_n, cnt_p, cnt_n, hcat, wp, wn, bp, bn)


def _lstm_from(eb, w_ih_t, w_hh_t, bias):
    cells = w_ih_t.shape[0]
    h = jnp.zeros((eb.shape[0], 64), _f32)
    c = jnp.zeros((eb.shape[0], 64), _f32)
    for i in range(cells):
        gates = (jnp.dot(eb, w_ih_t[i], preferred_element_type=_f32)
                 + jnp.dot(h, w_hh_t[i], preferred_element_type=_f32)
                 + bias[i])
        ig = jax.nn.sigmoid(gates[:, :64])
        fg = jax.nn.sigmoid(gates[:, 64:128])
        gg = jnp.tanh(gates[:, 128:192])
        og = jax.nn.sigmoid(gates[:, 192:256])
        c = fg * c + ig * gg
        h = og * jnp.tanh(c)
    return h


def _deep_lstm_body(s_p, s_n, cnt_p, cnt_n, hcat, wp, wn, bp, bn,
                    w_ih_t, w_hh_t, bias, out):
    emb = _deep_emb(s_p, s_n, cnt_p, cnt_n, hcat, wp, wn, bp, bn)
    out[...] = _lstm_from(emb, w_ih_t[...], w_hh_t[...], bias[...])


def kernel(x, pos_edge_index, neg_edge_index, W_pos_base, b_pos_base,
           W_neg_base, b_neg_base, W_pos_deep, b_pos_deep, W_neg_deep,
           b_neg_deep, W_ih, W_hh, b_ih, b_hh):
    n, d_feat = x.shape
    e = pos_edge_index.shape[1]
    hid = W_pos_base.shape[1]
    dcat = 2 * hid

    n_pad = ((n + R_BLOCK - 1) // R_BLOCK) * R_BLOCK
    grid = n_pad // R_BLOCK
    ck = 128
    dproj = hid + 16
    quantum = NUM_TILES * IB * ck
    e_pad = ((e + quantum - 1) // quantum) * quantum
    chunks = e_pad // (NUM_TILES * ck)

    src_p, dst_p = _pad_edges(pos_edge_index, e_pad, n_pad - 1, ck)
    src_n, dst_n = _pad_edges(neg_edge_index, e_pad, n_pad - 1, ck)
    src_d = jnp.stack([src_p, src_n])
    dst_d = jnp.stack([dst_p, dst_n])
    src_b = jnp.stack([src_p, src_n + n_pad])

    rpt = n_pad // NUM_TILES
    zrow_base = jnp.zeros((rpt, dproj), _f32)
    zrow_deep = jnp.zeros((rpt, dcat), _f32)

    x_pad = jnp.pad(x, ((0, n_pad - n), (0, 0)))

    w_agg = jnp.stack([W_pos_base[:d_feat], W_neg_base[:d_feat]])
    ytab = pl.pallas_call(
        _proj_body,
        grid=(2 * grid,),
        in_specs=[
            pl.BlockSpec((R_BLOCK, d_feat), lambda i: (i % grid, 0)),
            pl.BlockSpec((1, d_feat, hid), lambda i: (i // grid, 0, 0)),
        ],
        out_specs=pl.BlockSpec((R_BLOCK, dproj), lambda i: (i, 0)),
        out_shape=jax.ShapeDtypeStruct((2 * n_pad, dproj), _f32),
    )(x_pad, w_agg)

    seg_base = _make_seg_sum(n_pad, dproj, ck, chunks)
    sums = seg_base(ytab, src_b, dst_d, zrow_base)

    bp = b_pos_base.reshape(1, hid)
    bn = b_neg_base.reshape(1, hid)
    hcat = pl.pallas_call(
        _base_dense_body,
        grid=(grid,),
        in_specs=[
            _pick_spec(0, R_BLOCK, dproj), _pick_spec(1, R_BLOCK, dproj),
            _row_spec(R_BLOCK, d_feat),
            _full_spec((d_feat, hid)), _full_spec((d_feat, hid)),
            _full_spec(bp.shape), _full_spec(bn.shape),
        ],
        out_specs=_row_spec(R_BLOCK, dcat),
        out_shape=jax.ShapeDtypeStruct((n_pad, dcat), _f32),
    )(sums, sums, x_pad, W_pos_base[d_feat:], W_neg_base[d_feat:], bp, bn)

    seg_deep = _make_seg_sum(n_pad, dcat, ck, chunks)
    w_ih_t = W_ih.transpose(0, 2, 1)
    w_hh_t = W_hh.transpose(0, 2, 1)
    bias = (b_ih + b_hh).reshape(W_ih.shape[0], 1, W_ih.shape[1])
    layers = W_pos_deep.shape[0]
    for i in range(layers):
        dsums = seg_deep(hcat, src_d, dst_d, zrow_deep)
        bpi = b_pos_deep[i].reshape(1, hid)
        bni = b_neg_deep[i].reshape(1, hid)
        last = i == layers - 1
        in_specs = [
            _pick_spec(0, R_BLOCK, dcat), _pick_spec(1, R_BLOCK, dcat),
            _pick_spec(0, R_BLOCK, dproj), _pick_spec(1, R_BLOCK, dproj),
            _row_spec(R_BLOCK, dcat),
            _full_spec(W_pos_deep[i].shape),
            _full_spec(W_neg_deep[i].shape),
            _full_spec(bpi.shape), _full_spec(bni.shape),
        ]
        args = (dsums, dsums, sums, sums, hcat, W_pos_deep[i],
                W_neg_deep[i], bpi, bni)
        if last:
            in_specs += [_full_spec(w_ih_t.shape), _full_spec(w_hh_t.shape),
                         _full_spec(bias.shape)]
            args += (w_ih_t, w_hh_t, bias)
        hcat = pl.pallas_call(
            _deep_lstm_body if last else _deep_dense_body,
            grid=(grid,),
            in_specs=in_specs,
            out_specs=_row_spec(R_BLOCK, dcat),
            out_shape=jax.ShapeDtypeStruct((n_pad, dcat), _f32),
        )(*args)

    return hcat[:n]


# speedup vs baseline: 9.8066x; 1.0297x over previous
import functools

import jax
import jax.numpy as jnp
from jax import lax
from jax.experimental import pallas as pl
from jax.experimental.pallas import tpu as pltpu
from jax.experimental.pallas import tpu_sc as plsc

NUM_CORES = 2
NUM_TILES = 16
NBUF = 8
IB = 16
R_BLOCK = 1024

_f32 = jnp.float32
_i32 = jnp.int32


@functools.cache
def _make_seg_sum(n_pad: int, d: int, ck: int, chunks: int):
    rpt = n_pad // NUM_TILES
    nblocks = chunks // IB
    groups = IB // NBUF

    mesh = plsc.VectorSubcoreMesh(
        core_axis_name="c", subcore_axis_name="s",
        num_cores=NUM_CORES, num_subcores=NUM_TILES)

    scratch = [
        pltpu.VMEM((IB, ck), _i32),
        pltpu.VMEM((IB, ck), _i32),
    ]
    scratch += [pltpu.VMEM((ck, d), _f32) for _ in range(NBUF)]
    scratch += [pltpu.SemaphoreType.DMA for _ in range(2 * NBUF)]
    scratch.append(pltpu.VMEM_SHARED((n_pad, d), _f32))

    @functools.partial(
        pl.kernel,
        out_type=jax.ShapeDtypeStruct((NUM_CORES, n_pad, d), _f32),
        mesh=mesh, scratch_types=scratch,
        compiler_params=pltpu.CompilerParams(use_tc_tiling_on_sc=False))
    def body(table_hbm, src_hbm, dst_hbm, zrow_hbm, out_sum,
             src_v, dst_v, *rest):
        rows = rest[:NBUF]
        gsem = rest[NBUF:2 * NBUF]
        ssem = rest[2 * NBUF:3 * NBUF]
        (acc_sh,) = rest[3 * NBUF:]

        cid = lax.axis_index("c")
        sid = lax.axis_index("s")

        pltpu.sync_copy(zrow_hbm, acc_sh.at[pl.ds(sid * rpt, rpt)])
        plsc.subcore_barrier()

        def drain_scatter(j):
            pltpu.make_async_copy(
                rows[j], acc_sh.at[dst_v.at[j]], ssem[j]).wait()

        def block_body(b, carry):
            @pl.when(b > 0)
            def _():
                for j in range(NBUF):
                    drain_scatter(j)

            base = sid * chunks + b * IB
            pltpu.sync_copy(src_hbm.at[cid, pl.ds(base, IB)], src_v)
            pltpu.sync_copy(dst_hbm.at[cid, pl.ds(base, IB)], dst_v)

            def group_body(g, carry2):
                @pl.when(g > 0)
                def _():
                    for j in range(NBUF):
                        drain_scatter(j)
                for j in range(NBUF):
                    pltpu.async_copy(
                        table_hbm.at[src_v.at[g * NBUF + j]], rows[j],
                        gsem[j])
                for j in range(NBUF):
                    c = g * NBUF + j
                    pltpu.make_async_copy(
                        table_hbm.at[src_v.at[c]], rows[j], gsem[j]).wait()
                    pltpu.async_copy(
                        rows[j], acc_sh.at[dst_v.at[c]], ssem[j], add=True)
                return carry2

            return lax.fori_loop(0, groups, group_body, carry)

        lax.fori_loop(0, nblocks, block_body, 0)
        for j in range(NBUF):
            drain_scatter(j)
        plsc.subcore_barrier()

        sl = pl.ds(sid * rpt, rpt)
        pltpu.sync_copy(acc_sh.at[sl], out_sum.at[cid, sl])

    return body


def _pad_edges(edge_index, e_pad, dummy_dst, ck):
    e = edge_index.shape[1]
    src = jnp.concatenate(
        [edge_index[0], jnp.zeros((e_pad - e,), _i32)]).reshape(-1, ck)
    dst = jnp.concatenate(
        [edge_index[1], jnp.full((e_pad - e,), dummy_dst, _i32)]
    ).reshape(-1, ck)
    return src, dst


def _row_spec(r, cols):
    return pl.BlockSpec((r, cols), lambda i: (i, 0))


def _pick_spec(which, r, cols):
    return pl.BlockSpec((1, r, cols), lambda i, w=which: (w, i, 0))


def _full_spec(shape):
    nd = len(shape)
    return pl.BlockSpec(shape, lambda i: (0,) * nd)


def _recip(sums_ref):
    return 1.0 / jnp.maximum(sums_ref[0][:, 32:33], 1.0)


def _proj_body(x, wa, y):
    xb = x[...]
    r = xb.shape[0]
    one = jnp.ones((r, 1), _f32)
    pad = jnp.zeros((r, 15), _f32)
    y[...] = jnp.concatenate(
        [jnp.dot(xb, wa[0], preferred_element_type=_f32), one, pad], 1)


def _base_dense_body(sum_p, sum_n, x, wpx, wnx, bp, bn, out):
    agg_p = sum_p[0][:, :32] * _recip(sum_p)
    agg_n = sum_n[0][:, :32] * _recip(sum_n)
    xb = x[...]
    hp = jnp.tanh(
        agg_p + jnp.dot(xb, wpx[...], preferred_element_type=_f32) + bp[...])
    hn = jnp.tanh(
        agg_n + jnp.dot(xb, wnx[...], preferred_element_type=_f32) + bn[...])
    out[...] = jnp.concatenate([hp, hn], axis=1)


def _deep_emb(s_p, s_n, cnt_p, cnt_n, hcat, wp, wn, bp, bn):
    rp = _recip(cnt_p)
    rn = _recip(cnt_n)
    sp = s_p[0]
    sn = s_n[0]
    hb = hcat[...]
    hp, hn = hb[:, :32], hb[:, 32:]
    feat = jnp.concatenate([
        sp[:, :32] * rp,
        sn[:, 32:] * rn,
        sp[:, 32:] * rp,
        sn[:, :32] * rn,
        hp, hn, 0.5 * (hp + hn),
    ], axis=1)
    np_ = jnp.tanh(jnp.dot(feat, wp[...], preferred_element_type=_f32) + bp[...])
    nn_ = jnp.tanh(jnp.dot(feat, wn[...], preferred_element_type=_f32) + bn[...])
    return jnp.concatenate([np_, nn_], axis=1)


def _deep_dense_body(s_p, s_n, cnt_p, cnt_n, hcat, wp, wn, bp, bn, out):
    out[...] = _deep_emb(s_p, s_n, cnt_p, cnt_n, hcat, wp, wn, bp, bn)


def _lstm_from(eb, w_ih_t, w_hh_t, bias):
    cells = w_ih_t.shape[0]
    h = jnp.zeros((eb.shape[0], 64), _f32)
    c = jnp.zeros((eb.shape[0], 64), _f32)
    for i in range(cells):
        gates = (jnp.dot(eb, w_ih_t[i], preferred_element_type=_f32)
                 + jnp.dot(h, w_hh_t[i], preferred_element_type=_f32)
                 + bias[i])
        ig = jax.nn.sigmoid(gates[:, :64])
        fg = jax.nn.sigmoid(gates[:, 64:128])
        gg = jnp.tanh(gates[:, 128:192])
        og = jax.nn.sigmoid(gates[:, 192:256])
        c = fg * c + ig * gg
        h = og * jnp.tanh(c)
    return h


def _deep_lstm_body(s_p, s_n, cnt_p, cnt_n, hcat, wp, wn, bp, bn,
                    w_ih_t, w_hh_t, bias, out):
    emb = _deep_emb(s_p, s_n, cnt_p, cnt_n, hcat, wp, wn, bp, bn)
    out[...] = _lstm_from(emb, w_ih_t[...], w_hh_t[...], bias[...])


def kernel(x, pos_edge_index, neg_edge_index, W_pos_base, b_pos_base,
           W_neg_base, b_neg_base, W_pos_deep, b_pos_deep, W_neg_deep,
           b_neg_deep, W_ih, W_hh, b_ih, b_hh):
    n, d_feat = x.shape
    e = pos_edge_index.shape[1]
    hid = W_pos_base.shape[1]
    dcat = 2 * hid

    n_pad = ((n + R_BLOCK - 1) // R_BLOCK) * R_BLOCK
    grid = n_pad // R_BLOCK
    ck = 128
    dproj = hid + 16
    quantum = NUM_TILES * IB * ck
    e_pad = ((e + quantum - 1) // quantum) * quantum
    chunks = e_pad // (NUM_TILES * ck)

    src_p, dst_p = _pad_edges(pos_edge_index, e_pad, n_pad - 1, ck)
    src_n, dst_n = _pad_edges(neg_edge_index, e_pad, n_pad - 1, ck)
    src_d = jnp.stack([src_p, src_n])
    dst_d = jnp.stack([dst_p, dst_n])
    src_b = jnp.stack([src_p, src_n + n_pad])

    rpt = n_pad // NUM_TILES
    zrow_base = jnp.zeros((rpt, dproj), _f32)
    zrow_deep = jnp.zeros((rpt, dcat), _f32)

    x_pad = jnp.pad(x, ((0, n_pad - n), (0, 0)))

    w_agg = jnp.stack([W_pos_base[:d_feat], W_neg_base[:d_feat]])
    ytab = pl.pallas_call(
        _proj_body,
        grid=(2 * grid,),
        in_specs=[
            pl.BlockSpec((R_BLOCK, d_feat), lambda i: (i % grid, 0)),
            pl.BlockSpec((1, d_feat, hid), lambda i: (i // grid, 0, 0)),
        ],
        out_specs=pl.BlockSpec((R_BLOCK, dproj), lambda i: (i, 0)),
        out_shape=jax.ShapeDtypeStruct((2 * n_pad, dproj), _f32),
    )(x_pad, w_agg)

    seg_base = _make_seg_sum(n_pad, dproj, ck, chunks)
    sums = seg_base(ytab, src_b, dst_d, zrow_base)

    bp = b_pos_base.reshape(1, hid)
    bn = b_neg_base.reshape(1, hid)
    hcat = pl.pallas_call(
        _base_dense_body,
        grid=(grid,),
        in_specs=[
            _pick_spec(0, R_BLOCK, dproj), _pick_spec(1, R_BLOCK, dproj),
            _row_spec(R_BLOCK, d_feat),
            _full_spec((d_feat, hid)), _full_spec((d_feat, hid)),
            _full_spec(bp.shape), _full_spec(bn.shape),
        ],
        out_specs=_row_spec(R_BLOCK, dcat),
        out_shape=jax.ShapeDtypeStruct((n_pad, dcat), _f32),
    )(sums, sums, x_pad, W_pos_base[d_feat:], W_neg_base[d_feat:], bp, bn)

    seg_deep = _make_seg_sum(n_pad, dcat, ck, chunks)
    w_ih_t = W_ih.transpose(0, 2, 1)
    w_hh_t = W_hh.transpose(0, 2, 1)
    bias = (b_ih + b_hh).reshape(W_ih.shape[0], 1, W_ih.shape[1])
    layers = W_pos_deep.shape[0]
    for i in range(layers):
        dsums = seg_deep(hcat, src_d, dst_d, zrow_deep)
        bpi = b_pos_deep[i].reshape(1, hid)
        bni = b_neg_deep[i].reshape(1, hid)
        last = i == layers - 1
        in_specs = [
            _pick_spec(0, R_BLOCK, dcat), _pick_spec(1, R_BLOCK, dcat),
            _pick_spec(0, R_BLOCK, dproj), _pick_spec(1, R_BLOCK, dproj),
            _row_spec(R_BLOCK, dcat),
            _full_spec(W_pos_deep[i].shape),
            _full_spec(W_neg_deep[i].shape),
            _full_spec(bpi.shape), _full_spec(bni.shape),
        ]
        args = (dsums, dsums, sums, sums, hcat, W_pos_deep[i],
                W_neg_deep[i], bpi, bni)
        if last:
            in_specs += [_full_spec(w_ih_t.shape), _full_spec(w_hh_t.shape),
                         _full_spec(bias.shape)]
            args += (w_ih_t, w_hh_t, bias)
        hcat = pl.pallas_call(
            _deep_lstm_body if last else _deep_dense_body,
            grid=(grid,),
            in_specs=in_specs,
            out_specs=_row_spec(R_BLOCK, dcat),
            out_shape=jax.ShapeDtypeStruct((n_pad, dcat), _f32),
        )(*args)

    return hcat[:n]


# speedup vs baseline: 10.2795x; 1.0482x over previous
import functools

import jax
import jax.numpy as jnp
from jax import lax
from jax.experimental import pallas as pl
from jax.experimental.pallas import tpu as pltpu
from jax.experimental.pallas import tpu_sc as plsc

NUM_CORES = 2
NUM_TILES = 16
NBUF = 8
IB = 32
R_BLOCK = 1024

_f32 = jnp.float32
_i32 = jnp.int32


@functools.cache
def _make_seg_sum(n_pad: int, d: int, ck: int, chunks: int):
    rpt = n_pad // NUM_TILES
    nblocks = chunks // IB
    groups = IB // NBUF

    mesh = plsc.VectorSubcoreMesh(
        core_axis_name="c", subcore_axis_name="s",
        num_cores=NUM_CORES, num_subcores=NUM_TILES)

    scratch = [
        pltpu.VMEM((IB, ck), _i32),
        pltpu.VMEM((IB, ck), _i32),
    ]
    scratch += [pltpu.VMEM((ck, d), _f32) for _ in range(NBUF)]
    scratch += [pltpu.SemaphoreType.DMA for _ in range(2 * NBUF)]
    scratch.append(pltpu.VMEM_SHARED((n_pad, d), _f32))

    @functools.partial(
        pl.kernel,
        out_type=jax.ShapeDtypeStruct((NUM_CORES, n_pad, d), _f32),
        mesh=mesh, scratch_types=scratch,
        compiler_params=pltpu.CompilerParams(use_tc_tiling_on_sc=False))
    def body(table_hbm, src_hbm, dst_hbm, zrow_hbm, out_sum,
             src_v, dst_v, *rest):
        rows = rest[:NBUF]
        gsem = rest[NBUF:2 * NBUF]
        ssem = rest[2 * NBUF:3 * NBUF]
        (acc_sh,) = rest[3 * NBUF:]

        cid = lax.axis_index("c")
        sid = lax.axis_index("s")

        pltpu.sync_copy(zrow_hbm, acc_sh.at[pl.ds(sid * rpt, rpt)])
        plsc.subcore_barrier()

        def drain_scatter(j):
            pltpu.make_async_copy(
                rows[j], acc_sh.at[dst_v.at[j]], ssem[j]).wait()

        def block_body(b, carry):
            @pl.when(b > 0)
            def _():
                for j in range(NBUF):
                    drain_scatter(j)

            base = sid * chunks + b * IB
            pltpu.sync_copy(src_hbm.at[cid, pl.ds(base, IB)], src_v)
            pltpu.sync_copy(dst_hbm.at[cid, pl.ds(base, IB)], dst_v)

            def group_body(g, carry2):
                for j in range(NBUF):
                    @pl.when(g > 0)
                    def _():
                        drain_scatter(j)
                    pltpu.async_copy(
                        table_hbm.at[src_v.at[g * NBUF + j]], rows[j],
                        gsem[j])
                for j in range(NBUF):
                    c = g * NBUF + j
                    pltpu.make_async_copy(
                        table_hbm.at[src_v.at[c]], rows[j], gsem[j]).wait()
                    pltpu.async_copy(
                        rows[j], acc_sh.at[dst_v.at[c]], ssem[j], add=True)
                return carry2

            return lax.fori_loop(0, groups, group_body, carry)

        lax.fori_loop(0, nblocks, block_body, 0)
        for j in range(NBUF):
            drain_scatter(j)
        plsc.subcore_barrier()

        sl = pl.ds(sid * rpt, rpt)
        pltpu.sync_copy(acc_sh.at[sl], out_sum.at[cid, sl])

    return body


def _pad_edges(edge_index, e_pad, dummy_dst, ck):
    e = edge_index.shape[1]
    src = jnp.concatenate(
        [edge_index[0], jnp.zeros((e_pad - e,), _i32)]).reshape(-1, ck)
    dst = jnp.concatenate(
        [edge_index[1], jnp.full((e_pad - e,), dummy_dst, _i32)]
    ).reshape(-1, ck)
    return src, dst


def _row_spec(r, cols):
    return pl.BlockSpec((r, cols), lambda i: (i, 0))


def _pick_spec(which, r, cols):
    return pl.BlockSpec((1, r, cols), lambda i, w=which: (w, i, 0))


def _full_spec(shape):
    nd = len(shape)
    return pl.BlockSpec(shape, lambda i: (0,) * nd)


def _recip(sums_ref):
    return 1.0 / jnp.maximum(sums_ref[0][:, 32:33], 1.0)


def _proj_body(x, wa, y):
    xb = x[...]
    r = xb.shape[0]
    one = jnp.ones((r, 1), _f32)
    pad = jnp.zeros((r, 15), _f32)
    y[...] = jnp.concatenate(
        [jnp.dot(xb, wa[0], preferred_element_type=_f32), one, pad], 1)


def _base_dense_body(sum_p, sum_n, x, wpx, wnx, bp, bn, out):
    agg_p = sum_p[0][:, :32] * _recip(sum_p)
    agg_n = sum_n[0][:, :32] * _recip(sum_n)
    xb = x[...]
    hp = jnp.tanh(
        agg_p + jnp.dot(xb, wpx[...], preferred_element_type=_f32) + bp[...])
    hn = jnp.tanh(
        agg_n + jnp.dot(xb, wnx[...], preferred_element_type=_f32) + bn[...])
    out[...] = jnp.concatenate([hp, hn], axis=1)


def _deep_emb(s_p, s_n, cnt_p, cnt_n, hcat, wp, wn, bp, bn):
    rp = _recip(cnt_p)
    rn = _recip(cnt_n)
    sp = s_p[0]
    sn = s_n[0]
    hb = hcat[...]
    hp, hn = hb[:, :32], hb[:, 32:]
    feat = jnp.concatenate([
        sp[:, :32] * rp,
        sn[:, 32:] * rn,
        sp[:, 32:] * rp,
        sn[:, :32] * rn,
        hp, hn, 0.5 * (hp + hn),
    ], axis=1)
    np_ = jnp.tanh(jnp.dot(feat, wp[...], preferred_element_type=_f32) + bp[...])
    nn_ = jnp.tanh(jnp.dot(feat, wn[...], preferred_element_type=_f32) + bn[...])
    return jnp.concatenate([np_, nn_], axis=1)


def _deep_dense_body(s_p, s_n, cnt_p, cnt_n, hcat, wp, wn, bp, bn, out):
    out[...] = _deep_emb(s_p, s_n, cnt_p, cnt_n, hcat, wp, wn, bp, bn)


def _lstm_from(eb, w_ih_t, w_hh_t, bias):
    cells = w_ih_t.shape[0]
    h = jnp.zeros((eb.shape[0], 64), _f32)
    c = jnp.zeros((eb.shape[0], 64), _f32)
    for i in range(cells):
        gates = (jnp.dot(eb, w_ih_t[i], preferred_element_type=_f32)
                 + jnp.dot(h, w_hh_t[i], preferred_element_type=_f32)
                 + bias[i])
        ig = jax.nn.sigmoid(gates[:, :64])
        fg = jax.nn.sigmoid(gates[:, 64:128])
        gg = jnp.tanh(gates[:, 128:192])
        og = jax.nn.sigmoid(gates[:, 192:256])
        c = fg * c + ig * gg
        h = og * jnp.tanh(c)
    return h


def _deep_lstm_body(s_p, s_n, cnt_p, cnt_n, hcat, wp, wn, bp, bn,
                    w_ih_t, w_hh_t, bias, out):
    emb = _deep_emb(s_p, s_n, cnt_p, cnt_n, hcat, wp, wn, bp, bn)
    out[...] = _lstm_from(emb, w_ih_t[...], w_hh_t[...], bias[...])


def kernel(x, pos_edge_index, neg_edge_index, W_pos_base, b_pos_base,
           W_neg_base, b_neg_base, W_pos_deep, b_pos_deep, W_neg_deep,
           b_neg_deep, W_ih, W_hh, b_ih, b_hh):
    n, d_feat = x.shape
    e = pos_edge_index.shape[1]
    hid = W_pos_base.shape[1]
    dcat = 2 * hid

    n_pad = ((n + R_BLOCK - 1) // R_BLOCK) * R_BLOCK
    grid = n_pad // R_BLOCK
    ck = 128
    dproj = hid + 16
    quantum = NUM_TILES * IB * ck
    e_pad = ((e + quantum - 1) // quantum) * quantum
    chunks = e_pad // (NUM_TILES * ck)

    src_p, dst_p = _pad_edges(pos_edge_index, e_pad, n_pad - 1, ck)
    src_n, dst_n = _pad_edges(neg_edge_index, e_pad, n_pad - 1, ck)
    src_d = jnp.stack([src_p, src_n])
    dst_d = jnp.stack([dst_p, dst_n])
    src_b = jnp.stack([src_p, src_n + n_pad])

    rpt = n_pad // NUM_TILES
    zrow_base = jnp.zeros((rpt, dproj), _f32)
    zrow_deep = jnp.zeros((rpt, dcat), _f32)

    x_pad = jnp.pad(x, ((0, n_pad - n), (0, 0)))

    w_agg = jnp.stack([W_pos_base[:d_feat], W_neg_base[:d_feat]])
    ytab = pl.pallas_call(
        _proj_body,
        grid=(2 * grid,),
        in_specs=[
            pl.BlockSpec((R_BLOCK, d_feat), lambda i: (i % grid, 0)),
            pl.BlockSpec((1, d_feat, hid), lambda i: (i // grid, 0, 0)),
        ],
        out_specs=pl.BlockSpec((R_BLOCK, dproj), lambda i: (i, 0)),
        out_shape=jax.ShapeDtypeStruct((2 * n_pad, dproj), _f32),
    )(x_pad, w_agg)

    seg_base = _make_seg_sum(n_pad, dproj, ck, chunks)
    sums = seg_base(ytab, src_b, dst_d, zrow_base)

    bp = b_pos_base.reshape(1, hid)
    bn = b_neg_base.reshape(1, hid)
    hcat = pl.pallas_call(
        _base_dense_body,
        grid=(grid,),
        in_specs=[
            _pick_spec(0, R_BLOCK, dproj), _pick_spec(1, R_BLOCK, dproj),
            _row_spec(R_BLOCK, d_feat),
            _full_spec((d_feat, hid)), _full_spec((d_feat, hid)),
            _full_spec(bp.shape), _full_spec(bn.shape),
        ],
        out_specs=_row_spec(R_BLOCK, dcat),
        out_shape=jax.ShapeDtypeStruct((n_pad, dcat), _f32),
    )(sums, sums, x_pad, W_pos_base[d_feat:], W_neg_base[d_feat:], bp, bn)

    seg_deep = _make_seg_sum(n_pad, dcat, ck, chunks)
    w_ih_t = W_ih.transpose(0, 2, 1)
    w_hh_t = W_hh.transpose(0, 2, 1)
    bias = (b_ih + b_hh).reshape(W_ih.shape[0], 1, W_ih.shape[1])
    layers = W_pos_deep.shape[0]
    for i in range(layers):
        dsums = seg_deep(hcat, src_d, dst_d, zrow_deep)
        bpi = b_pos_deep[i].reshape(1, hid)
        bni = b_neg_deep[i].reshape(1, hid)
        last = i == layers - 1
        in_specs = [
            _pick_spec(0, R_BLOCK, dcat), _pick_spec(1, R_BLOCK, dcat),
            _pick_spec(0, R_BLOCK, dproj), _pick_spec(1, R_BLOCK, dproj),
            _row_spec(R_BLOCK, dcat),
            _full_spec(W_pos_deep[i].shape),
            _full_spec(W_neg_deep[i].shape),
            _full_spec(bpi.shape), _full_spec(bni.shape),
        ]
        args = (dsums, dsums, sums, sums, hcat, W_pos_deep[i],
                W_neg_deep[i], bpi, bni)
        if last:
            in_specs += [_full_spec(w_ih_t.shape), _full_spec(w_hh_t.shape),
                         _full_spec(bias.shape)]
            args += (w_ih_t, w_hh_t, bias)
        hcat = pl.pallas_call(
            _deep_lstm_body if last else _deep_dense_body,
            grid=(grid,),
            in_specs=in_specs,
            out_specs=_row_spec(R_BLOCK, dcat),
            out_shape=jax.ShapeDtypeStruct((n_pad, dcat), _f32),
        )(*args)

    return hcat[:n]


# speedup vs baseline: 10.3348x; 1.0054x over previous
import functools

import jax
import jax.numpy as jnp
from jax import lax
from jax.experimental import pallas as pl
from jax.experimental.pallas import tpu as pltpu
from jax.experimental.pallas import tpu_sc as plsc

NUM_CORES = 2
NUM_TILES = 16
NBUF = 8
IB = 40
R_BLOCK = 1024

_f32 = jnp.float32
_i32 = jnp.int32


@functools.cache
def _make_seg_sum(n_pad: int, d: int, ck: int, chunks: int):
    rpt = n_pad // NUM_TILES
    nblocks = chunks // IB
    groups = IB // NBUF

    mesh = plsc.VectorSubcoreMesh(
        core_axis_name="c", subcore_axis_name="s",
        num_cores=NUM_CORES, num_subcores=NUM_TILES)

    scratch = [
        pltpu.VMEM((IB, ck), _i32),
        pltpu.VMEM((IB, ck), _i32),
    ]
    scratch += [pltpu.VMEM((ck, d), _f32) for _ in range(NBUF)]
    scratch += [pltpu.SemaphoreType.DMA for _ in range(2 * NBUF)]
    scratch.append(pltpu.VMEM_SHARED((n_pad, d), _f32))

    @functools.partial(
        pl.kernel,
        out_type=jax.ShapeDtypeStruct((NUM_CORES, n_pad, d), _f32),
        mesh=mesh, scratch_types=scratch,
        compiler_params=pltpu.CompilerParams(use_tc_tiling_on_sc=False))
    def body(table_hbm, src_hbm, dst_hbm, zrow_hbm, out_sum,
             src_v, dst_v, *rest):
        rows = rest[:NBUF]
        gsem = rest[NBUF:2 * NBUF]
        ssem = rest[2 * NBUF:3 * NBUF]
        (acc_sh,) = rest[3 * NBUF:]

        cid = lax.axis_index("c")
        sid = lax.axis_index("s")

        pltpu.sync_copy(zrow_hbm, acc_sh.at[pl.ds(sid * rpt, rpt)])
        plsc.subcore_barrier()

        def drain_scatter(j):
            pltpu.make_async_copy(
                rows[j], acc_sh.at[dst_v.at[j]], ssem[j]).wait()

        def block_body(b, carry):
            @pl.when(b > 0)
            def _():
                for j in range(NBUF):
                    drain_scatter(j)

            base = sid * chunks + b * IB
            pltpu.sync_copy(src_hbm.at[cid, pl.ds(base, IB)], src_v)
            pltpu.sync_copy(dst_hbm.at[cid, pl.ds(base, IB)], dst_v)

            def group_body(g, carry2):
                for j in range(NBUF):
                    @pl.when(g > 0)
                    def _():
                        drain_scatter(j)
                    pltpu.async_copy(
                        table_hbm.at[src_v.at[g * NBUF + j]], rows[j],
                        gsem[j])
                for j in range(NBUF):
                    c = g * NBUF + j
                    pltpu.make_async_copy(
                        table_hbm.at[src_v.at[c]], rows[j], gsem[j]).wait()
                    pltpu.async_copy(
                        rows[j], acc_sh.at[dst_v.at[c]], ssem[j], add=True)
                return carry2

            return lax.fori_loop(0, groups, group_body, carry)

        lax.fori_loop(0, nblocks, block_body, 0)
        for j in range(NBUF):
            drain_scatter(j)
        plsc.subcore_barrier()

        sl = pl.ds(sid * rpt, rpt)
        pltpu.sync_copy(acc_sh.at[sl], out_sum.at[cid, sl])

    return body


def _pad_edges(edge_index, e_pad, dummy_dst, ck):
    e = edge_index.shape[1]
    src = jnp.concatenate(
        [edge_index[0], jnp.zeros((e_pad - e,), _i32)]).reshape(-1, ck)
    dst = jnp.concatenate(
        [edge_index[1], jnp.full((e_pad - e,), dummy_dst, _i32)]
    ).reshape(-1, ck)
    return src, dst


def _row_spec(r, cols):
    return pl.BlockSpec((r, cols), lambda i: (i, 0))


def _pick_spec(which, r, cols):
    return pl.BlockSpec((1, r, cols), lambda i, w=which: (w, i, 0))


def _full_spec(shape):
    nd = len(shape)
    return pl.BlockSpec(shape, lambda i: (0,) * nd)


def _recip(sums_ref):
    return 1.0 / jnp.maximum(sums_ref[0][:, 32:33], 1.0)


def _proj_body(x, wa, y):
    xb = x[...]
    r = xb.shape[0]
    one = jnp.ones((r, 1), _f32)
    pad = jnp.zeros((r, 15), _f32)
    y[...] = jnp.concatenate(
        [jnp.dot(xb, wa[0], preferred_element_type=_f32), one, pad], 1)


def _base_dense_body(sum_p, sum_n, x, wpx, wnx, bp, bn, out):
    agg_p = sum_p[0][:, :32] * _recip(sum_p)
    agg_n = sum_n[0][:, :32] * _recip(sum_n)
    xb = x[...]
    hp = jnp.tanh(
        agg_p + jnp.dot(xb, wpx[...], preferred_element_type=_f32) + bp[...])
    hn = jnp.tanh(
        agg_n + jnp.dot(xb, wnx[...], preferred_element_type=_f32) + bn[...])
    out[...] = jnp.concatenate([hp, hn], axis=1)


def _deep_emb(s_p, s_n, cnt_p, cnt_n, hcat, wp, wn, bp, bn):
    rp = _recip(cnt_p)
    rn = _recip(cnt_n)
    sp = s_p[0]
    sn = s_n[0]
    hb = hcat[...]
    hp, hn = hb[:, :32], hb[:, 32:]
    feat = jnp.concatenate([
        sp[:, :32] * rp,
        sn[:, 32:] * rn,
        sp[:, 32:] * rp,
        sn[:, :32] * rn,
        hp, hn, 0.5 * (hp + hn),
    ], axis=1)
    np_ = jnp.tanh(jnp.dot(feat, wp[...], preferred_element_type=_f32) + bp[...])
    nn_ = jnp.tanh(jnp.dot(feat, wn[...], preferred_element_type=_f32) + bn[...])
    return jnp.concatenate([np_, nn_], axis=1)


def _deep_dense_body(s_p, s_n, cnt_p, cnt_n, hcat, wp, wn, bp, bn, out):
    out[...] = _deep_emb(s_p, s_n, cnt_p, cnt_n, hcat, wp, wn, bp, bn)


def _lstm_from(eb, w_ih_t, w_hh_t, bias):
    cells = w_ih_t.shape[0]
    h = jnp.zeros((eb.shape[0], 64), _f32)
    c = jnp.zeros((eb.shape[0], 64), _f32)
    for i in range(cells):
        gates = (jnp.dot(eb, w_ih_t[i], preferred_element_type=_f32)
                 + jnp.dot(h, w_hh_t[i], preferred_element_type=_f32)
                 + bias[i])
        ig = jax.nn.sigmoid(gates[:, :64])
        fg = jax.nn.sigmoid(gates[:, 64:128])
        gg = jnp.tanh(gates[:, 128:192])
        og = jax.nn.sigmoid(gates[:, 192:256])
        c = fg * c + ig * gg
        h = og * jnp.tanh(c)
    return h


def _deep_lstm_body(s_p, s_n, cnt_p, cnt_n, hcat, wp, wn, bp, bn,
                    w_ih_t, w_hh_t, bias, out):
    emb = _deep_emb(s_p, s_n, cnt_p, cnt_n, hcat, wp, wn, bp, bn)
    out[...] = _lstm_from(emb, w_ih_t[...], w_hh_t[...], bias[...])


def kernel(x, pos_edge_index, neg_edge_index, W_pos_base, b_pos_base,
           W_neg_base, b_neg_base, W_pos_deep, b_pos_deep, W_neg_deep,
           b_neg_deep, W_ih, W_hh, b_ih, b_hh):
    n, d_feat = x.shape
    e = pos_edge_index.shape[1]
    hid = W_pos_base.shape[1]
    dcat = 2 * hid

    n_pad = ((n + R_BLOCK - 1) // R_BLOCK) * R_BLOCK
    grid = n_pad // R_BLOCK
    ck = 128
    dproj = hid + 16
    quantum = NUM_TILES * IB * ck
    e_pad = ((e + quantum - 1) // quantum) * quantum
    chunks = e_pad // (NUM_TILES * ck)

    src_p, dst_p = _pad_edges(pos_edge_index, e_pad, n_pad - 1, ck)
    src_n, dst_n = _pad_edges(neg_edge_index, e_pad, n_pad - 1, ck)
    src_d = jnp.stack([src_p, src_n])
    dst_d = jnp.stack([dst_p, dst_n])
    src_b = jnp.stack([src_p, src_n + n_pad])

    rpt = n_pad // NUM_TILES
    zrow_base = jnp.zeros((rpt, dproj), _f32)
    zrow_deep = jnp.zeros((rpt, dcat), _f32)

    x_pad = jnp.pad(x, ((0, n_pad - n), (0, 0)))

    w_agg = jnp.stack([W_pos_base[:d_feat], W_neg_base[:d_feat]])
    ytab = pl.pallas_call(
        _proj_body,
        grid=(2 * grid,),
        in_specs=[
            pl.BlockSpec((R_BLOCK, d_feat), lambda i: (i % grid, 0)),
            pl.BlockSpec((1, d_feat, hid), lambda i: (i // grid, 0, 0)),
        ],
        out_specs=pl.BlockSpec((R_BLOCK, dproj), lambda i: (i, 0)),
        out_shape=jax.ShapeDtypeStruct((2 * n_pad, dproj), _f32),
    )(x_pad, w_agg)

    seg_base = _make_seg_sum(n_pad, dproj, ck, chunks)
    sums = seg_base(ytab, src_b, dst_d, zrow_base)

    bp = b_pos_base.reshape(1, hid)
    bn = b_neg_base.reshape(1, hid)
    hcat = pl.pallas_call(
        _base_dense_body,
        grid=(grid,),
        in_specs=[
            _pick_spec(0, R_BLOCK, dproj), _pick_spec(1, R_BLOCK, dproj),
            _row_spec(R_BLOCK, d_feat),
            _full_spec((d_feat, hid)), _full_spec((d_feat, hid)),
            _full_spec(bp.shape), _full_spec(bn.shape),
        ],
        out_specs=_row_spec(R_BLOCK, dcat),
        out_shape=jax.ShapeDtypeStruct((n_pad, dcat), _f32),
    )(sums, sums, x_pad, W_pos_base[d_feat:], W_neg_base[d_feat:], bp, bn)

    seg_deep = _make_seg_sum(n_pad, dcat, ck, chunks)
    w_ih_t = W_ih.transpose(0, 2, 1)
    w_hh_t = W_hh.transpose(0, 2, 1)
    bias = (b_ih + b_hh).reshape(W_ih.shape[0], 1, W_ih.shape[1])
    layers = W_pos_deep.shape[0]
    for i in range(layers):
        dsums = seg_deep(hcat, src_d, dst_d, zrow_deep)
        bpi = b_pos_deep[i].reshape(1, hid)
        bni = b_neg_deep[i].reshape(1, hid)
        last = i == layers - 1
        in_specs = [
            _pick_spec(0, R_BLOCK, dcat), _pick_spec(1, R_BLOCK, dcat),
            _pick_spec(0, R_BLOCK, dproj), _pick_spec(1, R_BLOCK, dproj),
            _row_spec(R_BLOCK, dcat),
            _full_spec(W_pos_deep[i].shape),
            _full_spec(W_neg_deep[i].shape),
            _full_spec(bpi.shape), _full_spec(bni.shape),
        ]
        args = (dsums, dsums, sums, sums, hcat, W_pos_deep[i],
                W_neg_deep[i], bpi, bni)
        if last:
            in_specs += [_full_spec(w_ih_t.shape), _full_spec(w_hh_t.shape),
                         _full_spec(bias.shape)]
            args += (w_ih_t, w_hh_t, bias)
        hcat = pl.pallas_call(
            _deep_lstm_body if last else _deep_dense_body,
            grid=(grid,),
            in_specs=in_specs,
            out_specs=_row_spec(R_BLOCK, dcat),
            out_shape=jax.ShapeDtypeStruct((n_pad, dcat), _f32),
        )(*args)

    return hcat[:n]
